# Initial kernel scaffold; baseline (speedup 1.0000x reference)
import jax
import jax.numpy as jnp
from jax.experimental import pallas as pl


def kernel(pred_logits, pred_boxes, id_features, motions, target_sizes, track_idx):
    raise NotImplementedError("write your pallas kernel here")


# speedup vs baseline: 4.3875x; 4.3875x over previous
import functools

import jax
import jax.numpy as jnp
from jax import lax
from jax.experimental import pallas as pl
from jax.experimental.pallas import tpu as pltpu
from jax.experimental.pallas import tpu_sc as plsc

_Q = 20000
_C = 91
_D = 256
_K = 100
_KPAD = 128
_QPAD = 20480
_ROWS_BLK = 2000


def _rowmax_body(x_ref, o_ref):
    o_ref[...] = jnp.max(x_ref[...], axis=1, keepdims=True)


def _rowmax(logits2d):
    return pl.pallas_call(
        _rowmax_body,
        grid=(_Q // _ROWS_BLK,),
        in_specs=[pl.BlockSpec((_ROWS_BLK, _C), lambda i: (i, 0))],
        out_specs=pl.BlockSpec((_ROWS_BLK, 1), lambda i: (i, 0)),
        out_shape=jax.ShapeDtypeStruct((_Q, 1), jnp.float32),
    )(logits2d)


def _topk_rows_body(x_ref, idx_ref):
    vals = x_ref[...]
    r = lax.broadcasted_iota(jnp.int32, (_QPAD // 128, 128), 0)
    c = lax.broadcasted_iota(jnp.int32, (_QPAD // 128, 128), 1)
    flat = r * 128 + c
    lane = lax.broadcasted_iota(jnp.int32, (1, 128), 1)

    def body(k, carry):
        vals, out = carry
        m = jnp.max(vals)
        cand = jnp.where(vals == m, flat, jnp.int32(2**30))
        f = jnp.min(cand)
        out = jnp.where(lane == k, f, out)
        vals = jnp.where(flat == f, -jnp.inf, vals)
        return vals, out

    _, out = lax.fori_loop(
        0, _K, body, (vals, jnp.zeros((1, 128), jnp.int32))
    )
    idx_ref[...] = out


def _topk_rows(rowmax_pad):
    return pl.pallas_call(
        _topk_rows_body,
        out_shape=jax.ShapeDtypeStruct((1, 128), jnp.int32),
    )(rowmax_pad)


def _stage2_body(rows_sref, logit_ref, rowsdata_ref, box_ref, mot_ref,
                 trk_ref, scores_ref, labels_ref, grows_ref, small_ref,
                 scratch_ref, cand_ref):
    i = pl.program_id(0)

    @pl.when(i == 0)
    def _():
        scratch_ref[...] = jnp.full((_KPAD, _C), -jnp.inf, jnp.float32)
        cand_ref[...] = jnp.zeros((_KPAD, 8), jnp.float32)

    scratch_ref[pl.ds(i, 1), :] = logit_ref[0]
    row_small = jnp.concatenate(
        [box_ref[0], mot_ref[0], trk_ref[0].astype(jnp.float32),
         jnp.zeros((1, 1), jnp.float32)], axis=1)
    cand_ref[pl.ds(i, 1), :] = row_small

    @pl.when(i == _K - 1)
    def _():
        vals = scratch_ref[...]
        col = lax.broadcasted_iota(jnp.int32, (_KPAD, _C), 1)
        flat = rowsdata_ref[...] * _C + col
        lflat = (lax.broadcasted_iota(jnp.int32, (_KPAD, _C), 0) * _C
                 + col)
        lane = lax.broadcasted_iota(jnp.int32, (1, 128), 1)
        big = jnp.int32(2**30)

        def body(k, carry):
            vals, vsel, fsel, psel = carry
            m = jnp.max(vals)
            f = jnp.min(jnp.where(vals == m, flat, big))
            p = jnp.min(jnp.where(flat == f, lflat, big)) // _C
            vsel = jnp.where(lane == k, m, vsel)
            fsel = jnp.where(lane == k, f, fsel)
            psel = jnp.where(lane == k, p, psel)
            vals = jnp.where(flat == f, -jnp.inf, vals)
            return vals, vsel, fsel, psel

        _, vsel, fsel, psel = lax.fori_loop(
            0, _K, body,
            (vals, jnp.zeros((1, 128), jnp.float32),
             jnp.zeros((1, 128), jnp.int32),
             jnp.zeros((1, 128), jnp.int32)))
        scores_ref[...] = jax.nn.sigmoid(vsel)
        labels_ref[...] = fsel % _C
        grows_ref[...] = jnp.where(lane < _K, fsel // _C, 0)
        rowi = lax.broadcasted_iota(jnp.int32, (_KPAD, _KPAD), 0)
        pt = (rowi == psel).astype(jnp.float32)
        small_ref[...] = lax.dot_general(
            pt, cand_ref[...], (((0,), (0,)), ((), ())),
            precision=lax.Precision.HIGHEST,
            preferred_element_type=jnp.float32)


def _stage2(rows128, logits3d, rows2d, box3d, mot3d, trk3d):
    grid_spec = pltpu.PrefetchScalarGridSpec(
        num_scalar_prefetch=1,
        grid=(_K,),
        in_specs=[
            pl.BlockSpec((1, 1, _C), lambda i, rows: (rows[i], 0, 0)),
            pl.BlockSpec((_KPAD, 1), lambda i, rows: (0, 0)),
            pl.BlockSpec((1, 1, 4), lambda i, rows: (rows[i], 0, 0)),
            pl.BlockSpec((1, 1, 2), lambda i, rows: (rows[i], 0, 0)),
            pl.BlockSpec((1, 1, 1), lambda i, rows: (rows[i], 0, 0)),
        ],
        out_specs=[
            pl.BlockSpec((1, 128), lambda i, rows: (0, 0)),
            pl.BlockSpec((1, 128), lambda i, rows: (0, 0)),
            pl.BlockSpec((1, 128), lambda i, rows: (0, 0)),
            pl.BlockSpec((_KPAD, 8), lambda i, rows: (0, 0)),
        ],
        scratch_shapes=[
            pltpu.VMEM((_KPAD, _C), jnp.float32),
            pltpu.VMEM((_KPAD, 8), jnp.float32),
        ],
    )
    return pl.pallas_call(
        _stage2_body,
        grid_spec=grid_spec,
        out_shape=[
            jax.ShapeDtypeStruct((1, 128), jnp.float32),
            jax.ShapeDtypeStruct((1, 128), jnp.int32),
            jax.ShapeDtypeStruct((1, 128), jnp.int32),
            jax.ShapeDtypeStruct((_KPAD, 8), jnp.float32),
        ],
    )(rows128, logits3d, rows2d, box3d, mot3d, trk3d)


_RPW = 16


def _sc_gather_body(rows_hbm, id_hbm, out_id, idx_v, idbuf, sem):
    wid = lax.axis_index("s") * 2 + lax.axis_index("c")

    @pl.when(wid < 8)
    def _():
        base = wid * _RPW
        pltpu.sync_copy(rows_hbm.at[pl.ds(base, _RPW)], idx_v)
        pltpu.async_copy(id_hbm.at[idx_v], idbuf, sem).wait()
        pltpu.sync_copy(idbuf, out_id.at[pl.ds(base, _RPW)])


def _sc_gather(rows128, id2d):
    mesh = plsc.VectorSubcoreMesh(core_axis_name="c", subcore_axis_name="s")
    kfn = functools.partial(
        pl.kernel,
        mesh=mesh,
        out_type=jax.ShapeDtypeStruct((_KPAD, _D), jnp.float32),
        scratch_types=[
            pltpu.VMEM((_RPW,), jnp.int32),
            pltpu.VMEM((_RPW, _D), jnp.float32),
            pltpu.SemaphoreType.DMA,
        ],
    )(_sc_gather_body)
    return kfn(rows128, id2d)


def kernel(pred_logits, pred_boxes, id_features, motions, target_sizes,
           track_idx):
    B, Q, C = pred_logits.shape
    D = id_features.shape[-1]

    logits2d = pred_logits.reshape(Q, C)
    rowmax = _rowmax(logits2d).reshape(Q)
    rowmax_pad = jnp.concatenate(
        [rowmax, jnp.full((_QPAD - Q,), -jnp.inf, jnp.float32)]
    ).reshape(_QPAD // 128, 128)

    rows = _topk_rows(rowmax_pad)
    rows128 = rows.reshape(128)
    rows2d = rows.reshape(128, 1)
    logits3d = pred_logits.reshape(Q, 1, C)

    scores_p, labels_p, grows_p, small = _stage2(
        rows128, logits3d, rows2d, pred_boxes.reshape(Q, 1, 4),
        motions.reshape(Q, 1, 2), track_idx.reshape(Q, 1, 1))
    grows = grows_p.reshape(128)

    id_g = _sc_gather(grows, id_features.reshape(Q, D))

    scores = scores_p[:, :_K]
    labels = labels_p[:, :_K]

    img_h = target_sizes[:, 0]
    img_w = target_sizes[:, 1]
    scale_fct = jnp.stack([img_w, img_h, img_w, img_h], axis=1)

    bg = small[:_K, 0:4]
    cx, cy, w, h = bg[:, 0], bg[:, 1], bg[:, 2], bg[:, 3]
    boxes = jnp.stack(
        [cx - 0.5 * w, cy - 0.5 * h, cx + 0.5 * w, cy + 0.5 * h], axis=-1)
    boxes = (boxes[None] * scale_fct[:, None, :])

    id_out = id_g[:_K]
    mot_out = small[:_K, 4:6] * scale_fct[:, :2]
    trk_out = jnp.round(small[:_K, 6]).astype(jnp.int32)
    return scores, labels, boxes, id_out, mot_out, trk_out


# speedup vs baseline: 5.7008x; 1.2993x over previous
import functools

import jax
import jax.numpy as jnp
from jax import lax
from jax.experimental import pallas as pl
from jax.experimental.pallas import tpu as pltpu
from jax.experimental.pallas import tpu_sc as plsc

_Q = 20000
_C = 91
_D = 256
_K = 100
_KPAD = 128
_G = 50
_W = 400


def _rowmax_body(x_ref, o_ref, id_ref):
    i = pl.program_id(0)

    @pl.when(i == 0)
    def _():
        r = lax.broadcasted_iota(jnp.int32, (_W, _W), 0)
        c = lax.broadcasted_iota(jnp.int32, (_W, _W), 1)
        id_ref[...] = (r == c).astype(jnp.float32)

    rm = jnp.max(x_ref[...], axis=1, keepdims=True)
    rt = lax.dot_general(
        rm, id_ref[...], (((0,), (0,)), ((), ())),
        precision=lax.Precision.HIGHEST,
        preferred_element_type=jnp.float32)
    o_ref[...] = rt.reshape(1, 1, _W)


def _rowmax(logits2d):
    return pl.pallas_call(
        _rowmax_body,
        grid=(_G,),
        in_specs=[pl.BlockSpec((_W, _C), lambda i: (i, 0))],
        out_specs=pl.BlockSpec((1, 1, _W), lambda i: (i, 0, 0)),
        out_shape=jax.ShapeDtypeStruct((_G, 1, _W), jnp.float32),
        scratch_shapes=[pltpu.VMEM((_W, _W), jnp.float32)],
    )(logits2d)


def _select_body(rm_ref, logits_hbm, box_hbm, mot_hbm, trk_hbm,
                 scores_ref, labels_ref, grows_ref, small_ref,
                 vals_ref, gmax_ref, cand_ref, rowcol_ref,
                 cbox_ref, cmot_ref, ctrk_ref, rsm_ref, rsm2_ref,
                 seml, semb, semm, semt):
    big = jnp.int32(2**30)
    neg = -jnp.inf

    vals_ref[...] = rm_ref[...].reshape(_G, _W)
    gmax_ref[...] = jnp.max(vals_ref[...], axis=1, keepdims=True)
    cand_ref[...] = jnp.full((_KPAD, _C), neg, jnp.float32)
    rowcol_ref[...] = jnp.zeros((_KPAD, 1), jnp.int32)

    subl = lax.broadcasted_iota(jnp.int32, (_G, 1), 0)
    lanew = lax.broadcasted_iota(jnp.int32, (1, _W), 1)

    def pick(k, carry):
        m = jnp.max(gmax_ref[...])
        g = jnp.min(jnp.where(gmax_ref[...] == m, subl, big))
        row = vals_ref[pl.ds(g, 1), :]
        l = jnp.min(jnp.where(row == m, lanew, big))
        r = g * _W + l
        rsm_ref[k] = r
        rowcol_ref[pl.ds(k, 1), :] = jnp.full((1, 1), r, jnp.int32)
        row2 = jnp.where(lanew == l, neg, row)
        vals_ref[pl.ds(g, 1), :] = row2
        gmax_ref[pl.ds(g, 1), :] = jnp.max(row2, axis=1, keepdims=True)
        return carry

    lax.fori_loop(0, _K, pick, 0)

    copies = []
    for k in range(_K):
        cp = pltpu.make_async_copy(
            logits_hbm.at[pl.ds(rsm_ref[k], 1), :],
            cand_ref.at[pl.ds(k, 1), :], seml)
        cp.start()
        copies.append(cp)
    for cp in copies:
        cp.wait()

    col = lax.broadcasted_iota(jnp.int32, (_KPAD, _C), 1)
    flat = rowcol_ref[...] * _C + col
    lane = lax.broadcasted_iota(jnp.int32, (1, 128), 1)

    def body(k, carry):
        vals, vsel, fsel = carry
        m = jnp.max(vals)
        f = jnp.min(jnp.where(vals == m, flat, big))
        rsm2_ref[k] = f // _C
        vsel = jnp.where(lane == k, m, vsel)
        fsel = jnp.where(lane == k, f, fsel)
        vals = jnp.where(flat == f, neg, vals)
        return vals, vsel, fsel

    _, vsel, fsel = lax.fori_loop(
        0, _K, body,
        (cand_ref[...], jnp.zeros((1, 128), jnp.float32),
         jnp.zeros((1, 128), jnp.int32)))

    scores_ref[...] = jax.nn.sigmoid(vsel)
    labels_ref[...] = fsel % _C
    grows_ref[...] = jnp.where(lane < _K, fsel // _C, 0)

    copies = []
    for k in range(_K):
        r2 = rsm2_ref[k]
        for src, dst, sem in ((box_hbm, cbox_ref, semb),
                              (mot_hbm, cmot_ref, semm),
                              (trk_hbm, ctrk_ref, semt)):
            cp = pltpu.make_async_copy(
                src.at[pl.ds(r2, 1), :], dst.at[pl.ds(k, 1), :], sem)
            cp.start()
            copies.append(cp)
    for cp in copies:
        cp.wait()

    small_ref[...] = jnp.concatenate(
        [cbox_ref[...], cmot_ref[...], ctrk_ref[...].astype(jnp.float32),
         jnp.zeros((_KPAD, 1), jnp.float32)], axis=1)


def _select(rowmax3d, logits2d, box2d, mot2d, trk2d):
    return pl.pallas_call(
        _select_body,
        in_specs=[
            pl.BlockSpec((_G, 1, _W), lambda: (0, 0, 0)),
            pl.BlockSpec(memory_space=pl.ANY),
            pl.BlockSpec(memory_space=pl.ANY),
            pl.BlockSpec(memory_space=pl.ANY),
            pl.BlockSpec(memory_space=pl.ANY),
        ],
        out_shape=[
            jax.ShapeDtypeStruct((1, 128), jnp.float32),
            jax.ShapeDtypeStruct((1, 128), jnp.int32),
            jax.ShapeDtypeStruct((1, 128), jnp.int32),
            jax.ShapeDtypeStruct((_KPAD, 8), jnp.float32),
        ],
        scratch_shapes=[
            pltpu.VMEM((_G, _W), jnp.float32),
            pltpu.VMEM((_G, 1), jnp.float32),
            pltpu.VMEM((_KPAD, _C), jnp.float32),
            pltpu.VMEM((_KPAD, 1), jnp.int32),
            pltpu.VMEM((_KPAD, 4), jnp.float32),
            pltpu.VMEM((_KPAD, 2), jnp.float32),
            pltpu.VMEM((_KPAD, 1), jnp.int32),
            pltpu.SMEM((_K,), jnp.int32),
            pltpu.SMEM((_K,), jnp.int32),
            pltpu.SemaphoreType.DMA,
            pltpu.SemaphoreType.DMA,
            pltpu.SemaphoreType.DMA,
            pltpu.SemaphoreType.DMA,
        ],
    )(rowmax3d, logits2d, box2d, mot2d, trk2d)


_RPW = 16


def _sc_gather_body(rows_hbm, id_hbm, out_id, idx_v, idbuf, sem):
    wid = lax.axis_index("s") * 2 + lax.axis_index("c")

    @pl.when(wid < 8)
    def _():
        base = wid * _RPW
        pltpu.sync_copy(rows_hbm.at[pl.ds(base, _RPW)], idx_v)
        pltpu.async_copy(id_hbm.at[idx_v], idbuf, sem).wait()
        pltpu.sync_copy(idbuf, out_id.at[pl.ds(base, _RPW)])


def _sc_gather(rows128, id2d):
    mesh = plsc.VectorSubcoreMesh(core_axis_name="c", subcore_axis_name="s")
    kfn = functools.partial(
        pl.kernel,
        mesh=mesh,
        out_type=jax.ShapeDtypeStruct((_KPAD, _D), jnp.float32),
        scratch_types=[
            pltpu.VMEM((_RPW,), jnp.int32),
            pltpu.VMEM((_RPW, _D), jnp.float32),
            pltpu.SemaphoreType.DMA,
        ],
    )(_sc_gather_body)
    return kfn(rows128, id2d)


def kernel(pred_logits, pred_boxes, id_features, motions, target_sizes,
           track_idx):
    B, Q, C = pred_logits.shape
    D = id_features.shape[-1]

    logits2d = pred_logits.reshape(Q, C)
    rowmax3d = _rowmax(logits2d)

    scores_p, labels_p, grows_p, small = _select(
        rowmax3d, logits2d, pred_boxes.reshape(Q, 4),
        motions.reshape(Q, 2), track_idx.reshape(Q, 1))
    grows = grows_p.reshape(128)

    id_g = _sc_gather(grows, id_features.reshape(Q, D))

    scores = scores_p[:, :_K]
    labels = labels_p[:, :_K]

    img_h = target_sizes[:, 0]
    img_w = target_sizes[:, 1]
    scale_fct = jnp.stack([img_w, img_h, img_w, img_h], axis=1)

    bg = small[:_K, 0:4]
    cx, cy, w, h = bg[:, 0], bg[:, 1], bg[:, 2], bg[:, 3]
    boxes = jnp.stack(
        [cx - 0.5 * w, cy - 0.5 * h, cx + 0.5 * w, cy + 0.5 * h], axis=-1)
    boxes = (boxes[None] * scale_fct[:, None, :])

    id_out = id_g[:_K]
    mot_out = small[:_K, 4:6] * scale_fct[:, :2]
    trk_out = jnp.round(small[:_K, 6]).astype(jnp.int32)
    return scores, labels, boxes, id_out, mot_out, trk_out


# speedup vs baseline: 6.4973x; 1.1397x over previous
import functools

import jax
import jax.numpy as jnp
from jax import lax
from jax.experimental import pallas as pl
from jax.experimental.pallas import tpu as pltpu
from jax.experimental.pallas import tpu_sc as plsc

_Q = 20000
_C = 91
_D = 256
_K = 100
_KPAD = 128
_G = 50
_W = 400


def _select_body(logit_ref, box_ref, mot_ref, trk_ref, ts_ref,
                 scores_ref, labels_ref, grows_ref, boxes_ref, motout_ref,
                 trkout_ref, vals_ref, gmax_ref, cand_ref, rowcol_ref,
                 cbox_ref, cmot_ref, ctrk_ref):
    big = jnp.int32(2**30)
    neg = -jnp.inf

    for j in range(_G):
        rm = jnp.max(logit_ref[pl.ds(j * _W, _W), :], axis=1,
                     keepdims=True)
        vals_ref[pl.ds(j, 1), :] = rm.reshape(1, _W)

    gmax_ref[...] = jnp.max(vals_ref[...], axis=1, keepdims=True)
    cand_ref[...] = jnp.full((_KPAD, _C), neg, jnp.float32)
    rowcol_ref[...] = jnp.zeros((_KPAD, 1), jnp.int32)

    subl = lax.broadcasted_iota(jnp.int32, (_G, 1), 0)
    lanew = lax.broadcasted_iota(jnp.int32, (1, _W), 1)

    def pick(k, carry):
        m = jnp.max(gmax_ref[...])
        g = jnp.min(jnp.where(gmax_ref[...] == m, subl, big))
        row = vals_ref[pl.ds(g, 1), :]
        l = jnp.min(jnp.where(row == m, lanew, big))
        r = g * _W + l
        cand_ref[pl.ds(k, 1), :] = logit_ref[pl.ds(r, 1), :]
        rowcol_ref[pl.ds(k, 1), :] = jnp.full((1, 1), r, jnp.int32)
        row2 = jnp.where(lanew == l, neg, row)
        vals_ref[pl.ds(g, 1), :] = row2
        gmax_ref[pl.ds(g, 1), :] = jnp.max(row2, axis=1, keepdims=True)
        return carry

    lax.fori_loop(0, _K, pick, 0)

    col = lax.broadcasted_iota(jnp.int32, (_KPAD, _C), 1)
    flat = rowcol_ref[...] * _C + col
    lane = lax.broadcasted_iota(jnp.int32, (1, 128), 1)

    def body(k, carry):
        vals, vsel, fsel = carry
        m = jnp.max(vals)
        f = jnp.min(jnp.where(vals == m, flat, big))
        r2 = f // _C
        cbox_ref[pl.ds(k, 1), :] = box_ref[pl.ds(r2, 1), :]
        cmot_ref[pl.ds(k, 1), :] = mot_ref[pl.ds(r2, 1), :]
        ctrk_ref[pl.ds(k, 1), :] = trk_ref[pl.ds(r2, 1), :]
        vsel = jnp.where(lane == k, m, vsel)
        fsel = jnp.where(lane == k, f, fsel)
        vals = jnp.where(flat == f, neg, vals)
        return vals, vsel, fsel

    _, vsel, fsel = lax.fori_loop(
        0, _K, body,
        (cand_ref[...], jnp.zeros((1, 128), jnp.float32),
         jnp.zeros((1, 128), jnp.int32)))

    scores_ref[...] = jax.nn.sigmoid(vsel)
    labels_ref[...] = fsel % _C
    grows_ref[...] = jnp.where(lane < _K, fsel // _C, 0)

    ih = ts_ref[0]
    iw = ts_ref[1]
    cx = cbox_ref[:, 0:1]
    cy = cbox_ref[:, 1:2]
    w = cbox_ref[:, 2:3]
    h = cbox_ref[:, 3:4]
    boxes_ref[...] = jnp.concatenate(
        [(cx - 0.5 * w) * iw, (cy - 0.5 * h) * ih,
         (cx + 0.5 * w) * iw, (cy + 0.5 * h) * ih], axis=1)
    motout_ref[...] = jnp.concatenate(
        [cmot_ref[:, 0:1] * iw, cmot_ref[:, 1:2] * ih], axis=1)
    trkout_ref[...] = ctrk_ref[...]


def _select(logits2d, box2d, mot2d, trk2d, ts):
    return pl.pallas_call(
        _select_body,
        in_specs=[
            pl.BlockSpec((_Q, _C), lambda: (0, 0)),
            pl.BlockSpec((_Q, 4), lambda: (0, 0)),
            pl.BlockSpec((_Q, 2), lambda: (0, 0)),
            pl.BlockSpec((_Q, 1), lambda: (0, 0)),
            pl.BlockSpec(memory_space=pltpu.SMEM),
        ],
        out_shape=[
            jax.ShapeDtypeStruct((1, 128), jnp.float32),
            jax.ShapeDtypeStruct((1, 128), jnp.int32),
            jax.ShapeDtypeStruct((1, 128), jnp.int32),
            jax.ShapeDtypeStruct((_KPAD, 4), jnp.float32),
            jax.ShapeDtypeStruct((_KPAD, 2), jnp.float32),
            jax.ShapeDtypeStruct((_KPAD, 1), jnp.int32),
        ],
        scratch_shapes=[
            pltpu.VMEM((_G, _W), jnp.float32),
            pltpu.VMEM((_G, 1), jnp.float32),
            pltpu.VMEM((_KPAD, _C), jnp.float32),
            pltpu.VMEM((_KPAD, 1), jnp.int32),
            pltpu.VMEM((_KPAD, 4), jnp.float32),
            pltpu.VMEM((_KPAD, 2), jnp.float32),
            pltpu.VMEM((_KPAD, 1), jnp.int32),
        ],
    )(logits2d, box2d, mot2d, trk2d, ts)


_RPW = 16


def _sc_gather_body(rows_hbm, id_hbm, out_id, idx_v, idbuf, sem):
    wid = lax.axis_index("s") * 2 + lax.axis_index("c")

    @pl.when(wid < 8)
    def _():
        base = wid * _RPW
        pltpu.sync_copy(rows_hbm.at[pl.ds(base, _RPW)], idx_v)
        pltpu.async_copy(id_hbm.at[idx_v], idbuf, sem).wait()
        pltpu.sync_copy(idbuf, out_id.at[pl.ds(base, _RPW)])


def _sc_gather(rows128, id2d):
    mesh = plsc.VectorSubcoreMesh(core_axis_name="c", subcore_axis_name="s")
    kfn = functools.partial(
        pl.kernel,
        mesh=mesh,
        out_type=jax.ShapeDtypeStruct((_KPAD, _D), jnp.float32),
        scratch_types=[
            pltpu.VMEM((_RPW,), jnp.int32),
            pltpu.VMEM((_RPW, _D), jnp.float32),
            pltpu.SemaphoreType.DMA,
        ],
    )(_sc_gather_body)
    return kfn(rows128, id2d)


def kernel(pred_logits, pred_boxes, id_features, motions, target_sizes,
           track_idx):
    B, Q, C = pred_logits.shape
    D = id_features.shape[-1]

    scores_p, labels_p, grows_p, boxes_p, mot_p, trk_p = _select(
        pred_logits.reshape(Q, C), pred_boxes.reshape(Q, 4),
        motions.reshape(Q, 2), track_idx.reshape(Q, 1),
        target_sizes.reshape(2))
    grows = grows_p.reshape(128)

    id_g = _sc_gather(grows, id_features.reshape(Q, D))

    scores = scores_p[:, :_K]
    labels = labels_p[:, :_K]
    boxes = boxes_p[None, :_K, :]
    id_out = id_g[:_K]
    mot_out = mot_p[:_K]
    trk_out = trk_p[:_K, 0]
    return scores, labels, boxes, id_out, mot_out, trk_out
